# Initial kernel scaffold; baseline (speedup 1.0000x reference)
import functools

import numpy as np
import jax
import jax.numpy as jnp
from jax import lax
from jax.experimental import pallas as pl
from jax.experimental.pallas import tpu as pltpu
from jax.experimental.pallas import tpu_sc as plsc

_HASH_PRIMES = [2654435761, 2246822519, 3266489917, 2028178513, 1220703125,
                1610612741, 805306457, 402653189, 3674653429, 2860486313,
                1073676287, 2971215073, 1500450271, 3267000013, 2654435789,
                4049292737, 2246822531, 3266489927, 2028178519, 1220703133]

_B, _T = 16, 2048
_STATE = 256
_HEADS = 4
_BITS = 16
_BUCKETS = 131072
_KEYBITS = 17
_EMB = 32
_N = _B * _T
_NROWS = _N * _HEADS

_PBIG = np.zeros((_HEADS * _BITS, 128), np.float32)
_POW = np.zeros((128, _HEADS), np.float32)
for _h in range(_HEADS):
    for _i in range(_BITS):
        _p = _HASH_PRIMES[(_h * 3 + _i) % len(_HASH_PRIMES)] & (_BUCKETS - 1)
        for _j in range(_KEYBITS):
            if (_p >> _j) & 1:
                _PBIG[_h * _BITS + _i, _h * _KEYBITS + _j] = 1.0
    for _j in range(_KEYBITS):
        _POW[_h * _KEYBITS + _j, _h] = float(1 << _j)
_OFFS = (np.arange(_HEADS, dtype=np.float32) * _BUCKETS).reshape(1, _HEADS)

_BLK = 2048


def _keys_body(x_ref, wp_ref, bp_ref, pb_ref, pw_ref, off_ref, out_ref):
    x = x_ref[...]
    p = jnp.dot(x, wp_ref[...], preferred_element_type=jnp.float32)
    p = p + bp_ref[...]
    bits = (p > 0).astype(jnp.float32)
    counts = jnp.dot(bits, pb_ref[...],
                     preferred_element_type=jnp.float32)
    par = counts - 2.0 * jnp.floor(counts * 0.5)
    keyf = jnp.dot(par, pw_ref[...],
                   preferred_element_type=jnp.float32)
    out_ref[...] = (keyf + off_ref[...]).astype(jnp.int32)


def _compute_keys(x, Wp, bp):
    grid = _N // _BLK
    return pl.pallas_call(
        _keys_body,
        grid=(grid,),
        in_specs=[
            pl.BlockSpec((_BLK, _STATE), lambda i: (i, 0)),
            pl.BlockSpec((_STATE, _HEADS * _BITS), lambda i: (0, 0)),
            pl.BlockSpec((1, _HEADS * _BITS), lambda i: (0, 0)),
            pl.BlockSpec((_HEADS * _BITS, 128), lambda i: (0, 0)),
            pl.BlockSpec((128, _HEADS), lambda i: (0, 0)),
            pl.BlockSpec((1, _HEADS), lambda i: (0, 0)),
        ],
        out_specs=pl.BlockSpec((_BLK, _HEADS), lambda i: (i, 0)),
        out_shape=jax.ShapeDtypeStruct((_N, _HEADS), jnp.int32),
    )(x, Wp, bp.reshape(1, -1), jnp.asarray(_PBIG), jnp.asarray(_POW),
      jnp.asarray(_OFFS))


_IDX_COLS = 128
_IDX_ROWS = _NROWS // _IDX_COLS
_NW = 32
_ROWS_W = _IDX_ROWS // _NW


def _gather_body(table_hbm, idx_hbm, out_hbm, idx_v, buf_v, sem):
    wid = lax.axis_index("s") * 2 + lax.axis_index("c")
    base = wid * _ROWS_W
    pltpu.sync_copy(idx_hbm.at[pl.ds(base, _ROWS_W)], idx_v)

    def step(j, carry):
        pltpu.async_copy(table_hbm.at[idx_v.at[j]], buf_v, sem).wait()
        pltpu.sync_copy(
            buf_v, out_hbm.at[pl.ds((base + j) * _IDX_COLS, _IDX_COLS)])
        return carry

    lax.fori_loop(0, _ROWS_W, step, 0)


def _gather(flat_table, flat_idx):
    idx2d = flat_idx.reshape(_IDX_ROWS, _IDX_COLS)
    run = pl.kernel(
        _gather_body,
        mesh=plsc.VectorSubcoreMesh(core_axis_name="c", subcore_axis_name="s"),
        out_type=jax.ShapeDtypeStruct((_NROWS, _EMB), jnp.float32),
        scratch_types=[
            pltpu.VMEM((_ROWS_W, _IDX_COLS), jnp.int32),
            pltpu.VMEM((_IDX_COLS, _EMB), jnp.float32),
            pltpu.SemaphoreType.DMA,
        ],
    )
    return run(flat_table, idx2d)


def _out_body(e_ref, wo_ref, bo_ref, y_ref):
    y_ref[...] = jnp.dot(e_ref[...], wo_ref[...],
                         preferred_element_type=jnp.float32) + bo_ref[...]


def _out_proj(emb, Wo, bo):
    grid = _N // _BLK
    return pl.pallas_call(
        _out_body,
        grid=(grid,),
        in_specs=[
            pl.BlockSpec((_BLK, _HEADS * _EMB), lambda i: (i, 0)),
            pl.BlockSpec((_HEADS * _EMB, _STATE), lambda i: (0, 0)),
            pl.BlockSpec((1, _STATE), lambda i: (0, 0)),
        ],
        out_specs=pl.BlockSpec((_BLK, _STATE), lambda i: (i, 0)),
        out_shape=jax.ShapeDtypeStruct((_N, _STATE), jnp.float32),
    )(emb, Wo, bo.reshape(1, -1))


def kernel(scan_state, chars, Wp, bp, tables, Wo, bo):
    del chars
    x = scan_state.reshape(_N, _STATE)
    flat_idx = _compute_keys(x, Wp, bp).reshape(-1)
    flat_table = tables.reshape(_HEADS * _BUCKETS, _EMB)
    rows = _gather(flat_table, flat_idx)
    combined = rows.reshape(_N, _HEADS * _EMB)
    y = _out_proj(combined, Wo, bo)
    return y.reshape(_B, _T, _STATE)


# speedup vs baseline: 7.9531x; 7.9531x over previous
import functools

import numpy as np
import jax
import jax.numpy as jnp
from jax import lax
from jax.experimental import pallas as pl
from jax.experimental.pallas import tpu as pltpu
from jax.experimental.pallas import tpu_sc as plsc

_HASH_PRIMES = [2654435761, 2246822519, 3266489917, 2028178513, 1220703125,
                1610612741, 805306457, 402653189, 3674653429, 2860486313,
                1073676287, 2971215073, 1500450271, 3267000013, 2654435789,
                4049292737, 2246822531, 3266489927, 2028178519, 1220703133]

_B, _T = 16, 2048
_STATE = 256
_HEADS = 4
_BITS = 16
_BUCKETS = 131072
_KEYBITS = 17
_EMB = 32
_N = _B * _T
_NROWS = _N * _HEADS

_PBIG = np.zeros((_HEADS * _BITS, 128), np.float32)
_POW = np.zeros((128, _HEADS), np.float32)
for _h in range(_HEADS):
    for _i in range(_BITS):
        _p = _HASH_PRIMES[(_h * 3 + _i) % len(_HASH_PRIMES)] & (_BUCKETS - 1)
        for _j in range(_KEYBITS):
            if (_p >> _j) & 1:
                _PBIG[_h * _BITS + _i, _h * _KEYBITS + _j] = 1.0
    for _j in range(_KEYBITS):
        _POW[_h * _KEYBITS + _j, _h] = float(1 << _j)
_OFFS = (np.arange(_HEADS, dtype=np.float32) * _BUCKETS).reshape(1, _HEADS)

_BLK = 2048


def _Z():
    return jnp.int32(0)


def _keys_body(x_ref, wp_ref, bp_ref, pb_ref, pw_ref, off_ref, out_ref):
    x = x_ref[...]
    p = jnp.dot(x, wp_ref[...], preferred_element_type=jnp.float32)
    p = p + bp_ref[...]
    bits = (p > 0).astype(jnp.float32)
    counts = jnp.dot(bits, pb_ref[...],
                     preferred_element_type=jnp.float32)
    par = counts - 2.0 * jnp.floor(counts * 0.5)
    keyf = jnp.dot(par, pw_ref[...],
                   preferred_element_type=jnp.float32)
    out_ref[...] = (keyf + off_ref[...]).astype(jnp.int32)


def _compute_keys(x, Wp, bp):
    grid = _N // _BLK
    return pl.pallas_call(
        _keys_body,
        grid=(grid,),
        in_specs=[
            pl.BlockSpec((_BLK, _STATE), lambda i: (i, _Z())),
            pl.BlockSpec((_STATE, _HEADS * _BITS), lambda i: (_Z(), _Z())),
            pl.BlockSpec((1, _HEADS * _BITS), lambda i: (_Z(), _Z())),
            pl.BlockSpec((_HEADS * _BITS, 128), lambda i: (_Z(), _Z())),
            pl.BlockSpec((128, _HEADS), lambda i: (_Z(), _Z())),
            pl.BlockSpec((1, _HEADS), lambda i: (_Z(), _Z())),
        ],
        out_specs=pl.BlockSpec((_BLK, _HEADS), lambda i: (i, _Z())),
        out_shape=jax.ShapeDtypeStruct((_N, _HEADS), jnp.int32),
    )(x, Wp, bp.reshape(1, -1), jnp.asarray(_PBIG), jnp.asarray(_POW),
      jnp.asarray(_OFFS))


_IDX_COLS = 128
_IDX_ROWS = _NROWS // _IDX_COLS
_NW = 32
_ROWS_W = _IDX_ROWS // _NW


def _gather_body(table_hbm, idx_hbm, out_hbm, idx_v, buf_v, sem):
    wid = lax.axis_index("s") * 2 + lax.axis_index("c")
    base = wid * _ROWS_W
    pltpu.sync_copy(idx_hbm.at[pl.ds(base, _ROWS_W)], idx_v)

    def step(j, carry):
        pltpu.async_copy(table_hbm.at[idx_v.at[j]], buf_v, sem).wait()
        pltpu.sync_copy(
            buf_v, out_hbm.at[pl.ds((base + j) * _IDX_COLS, _IDX_COLS)])
        return carry

    lax.fori_loop(jnp.int32(0), jnp.int32(_ROWS_W), step, jnp.int32(0))


def _gather(flat_table, flat_idx):
    idx2d = flat_idx.reshape(_IDX_ROWS, _IDX_COLS)
    run = pl.kernel(
        _gather_body,
        mesh=plsc.VectorSubcoreMesh(core_axis_name="c", subcore_axis_name="s"),
        out_type=jax.ShapeDtypeStruct((_NROWS, _EMB), jnp.float32),
        scratch_types=[
            pltpu.VMEM((_ROWS_W, _IDX_COLS), jnp.int32),
            pltpu.VMEM((_IDX_COLS, _EMB), jnp.float32),
            pltpu.SemaphoreType.DMA,
        ],
        compiler_params=pltpu.CompilerParams(use_tc_tiling_on_sc=False),
    )
    return run(flat_table, idx2d)


def _out_body(e_ref, wo_ref, bo_ref, y_ref):
    y_ref[...] = jnp.dot(e_ref[...], wo_ref[...],
                         preferred_element_type=jnp.float32) + bo_ref[...]


def _out_proj(emb, Wo, bo):
    grid = _N // _BLK
    return pl.pallas_call(
        _out_body,
        grid=(grid,),
        in_specs=[
            pl.BlockSpec((_BLK, _HEADS * _EMB), lambda i: (i, _Z())),
            pl.BlockSpec((_HEADS * _EMB, _STATE), lambda i: (_Z(), _Z())),
            pl.BlockSpec((1, _STATE), lambda i: (_Z(), _Z())),
        ],
        out_specs=pl.BlockSpec((_BLK, _STATE), lambda i: (i, _Z())),
        out_shape=jax.ShapeDtypeStruct((_N, _STATE), jnp.float32),
    )(emb, Wo, bo.reshape(1, -1))


def kernel(scan_state, chars, Wp, bp, tables, Wo, bo):
    del chars
    x = scan_state.reshape(_N, _STATE)
    flat_idx = _compute_keys(x, Wp, bp).reshape(-1)
    flat_table = tables.reshape(_HEADS * _BUCKETS, _EMB)
    rows = _gather(flat_table, flat_idx)
    combined = rows.reshape(_N, _HEADS * _EMB)
    y = _out_proj(combined, Wo, bo)
    return y.reshape(_B, _T, _STATE)
